# Initial kernel scaffold; baseline (speedup 1.0000x reference)
import jax
import jax.numpy as jnp
from jax.experimental import pallas as pl


def kernel(x, edge_index, W1_l, b1_l, W1_r, W2_l, b2_l, W2_r):
    raise NotImplementedError("write your pallas kernel here")


# speedup vs baseline: 2.2460x; 2.2460x over previous
import functools

import jax
import jax.numpy as jnp
from jax import lax
from jax.experimental import pallas as pl
from jax.experimental.pallas import tpu as pltpu
from jax.experimental.pallas import tpu_sc as plsc

N = 10000
E = 320000
D = 128
NC = 2
NS = 16
NW = NC * NS
CPW = D // NW
LANES = 16
CHUNK = 3200
NCHUNK = E // CHUNK
GROUPS = CHUNK // LANES


def _sc_body(table_hbm, src_hbm, dst_hbm, out_hbm,
             xt, acc, deg, sb_a, sb_b, db_a, db_b,
             sem_tab, sem_as, sem_ad, sem_bs, sem_bd):
    cid = lax.axis_index("c")
    sid = lax.axis_index("s")
    wid = sid * NC + cid

    pltpu.make_async_copy(table_hbm.at[wid], xt, sem_tab).start()
    pltpu.make_async_copy(src_hbm.at[pl.ds(0, CHUNK)], sb_a, sem_as).start()
    pltpu.make_async_copy(dst_hbm.at[pl.ds(0, CHUNK)], db_a, sem_ad).start()

    zero16 = jnp.zeros((LANES,), jnp.float32)
    ones16 = jnp.ones((LANES,), jnp.float32)

    def zero_acc(i, carry):
        acc[pl.ds(i * LANES, LANES)] = zero16
        return carry

    lax.fori_loop(0, (N * CPW) // LANES, zero_acc, None)

    def zero_deg(i, carry):
        deg[pl.ds(i * LANES, LANES)] = zero16
        return carry

    lax.fori_loop(0, N // LANES, zero_deg, None)

    pltpu.make_async_copy(table_hbm.at[wid], xt, sem_tab).wait()

    def process(sb, db):
        def grp(g, carry):
            base = g * LANES
            s16 = sb[pl.ds(base, LANES)]
            d16 = db[pl.ds(base, LANES)]
            s4 = s16 * CPW
            d4 = d16 * CPW
            for c in range(CPW):
                vals = plsc.load_gather(xt, [s4 + c])
                plsc.addupdate_scatter(acc, [d4 + c], vals)
            plsc.addupdate_scatter(deg, [d16], ones16)
            return carry

        lax.fori_loop(0, GROUPS, grp, None)

    def chunk_pair(i, carry):
        c1 = 2 * i + 1
        pltpu.make_async_copy(src_hbm.at[pl.ds(c1 * CHUNK, CHUNK)], sb_b, sem_bs).start()
        pltpu.make_async_copy(dst_hbm.at[pl.ds(c1 * CHUNK, CHUNK)], db_b, sem_bd).start()
        pltpu.make_async_copy(src_hbm.at[pl.ds(0, CHUNK)], sb_a, sem_as).wait()
        pltpu.make_async_copy(dst_hbm.at[pl.ds(0, CHUNK)], db_a, sem_ad).wait()
        process(sb_a, db_a)
        c2 = ((2 * i + 2) % NCHUNK) * CHUNK
        pltpu.make_async_copy(src_hbm.at[pl.ds(c2, CHUNK)], sb_a, sem_as).start()
        pltpu.make_async_copy(dst_hbm.at[pl.ds(c2, CHUNK)], db_a, sem_ad).start()
        pltpu.make_async_copy(src_hbm.at[pl.ds(c1 * CHUNK, CHUNK)], sb_b, sem_bs).wait()
        pltpu.make_async_copy(dst_hbm.at[pl.ds(c1 * CHUNK, CHUNK)], db_b, sem_bd).wait()
        process(sb_b, db_b)
        return carry

    lax.fori_loop(0, NCHUNK // 2, chunk_pair, None)

    pltpu.make_async_copy(src_hbm.at[pl.ds(0, CHUNK)], sb_a, sem_as).wait()
    pltpu.make_async_copy(dst_hbm.at[pl.ds(0, CHUNK)], db_a, sem_ad).wait()

    node_rep = lax.iota(jnp.int32, LANES) >> 2

    def nrm(j, carry):
        dg = plsc.load_gather(deg, [j * 4 + node_rep])
        dg = jnp.maximum(dg, 1.0)
        a = acc[pl.ds(j * LANES, LANES)]
        acc[pl.ds(j * LANES, LANES)] = a / dg
        return carry

    lax.fori_loop(0, (N * CPW) // LANES, nrm, None)

    pltpu.sync_copy(acc, out_hbm.at[wid])


def _sc_segment_mean(table_r, src, dst):
    mesh = plsc.VectorSubcoreMesh(core_axis_name="c", subcore_axis_name="s")
    f = pl.kernel(
        _sc_body,
        out_type=jax.ShapeDtypeStruct((NW, N * CPW), jnp.float32),
        mesh=mesh,
        compiler_params=pltpu.CompilerParams(needs_layout_passes=False),
        scratch_types=[
            pltpu.VMEM((N * CPW,), jnp.float32),
            pltpu.VMEM((N * CPW,), jnp.float32),
            pltpu.VMEM((N,), jnp.float32),
            pltpu.VMEM((CHUNK,), jnp.int32),
            pltpu.VMEM((CHUNK,), jnp.int32),
            pltpu.VMEM((CHUNK,), jnp.int32),
            pltpu.VMEM((CHUNK,), jnp.int32),
            pltpu.SemaphoreType.DMA,
            pltpu.SemaphoreType.DMA,
            pltpu.SemaphoreType.DMA,
            pltpu.SemaphoreType.DMA,
            pltpu.SemaphoreType.DMA,
        ],
    )
    return f(table_r, src, dst)


def _dense(agg, xin, w_l, b_l, w_r, relu):
    bm = 1000

    def body(a_ref, x_ref, wl_ref, b_ref, wr_ref, o_ref):
        r = jnp.dot(a_ref[...], wl_ref[...], preferred_element_type=jnp.float32)
        r = r + jnp.dot(x_ref[...], wr_ref[...], preferred_element_type=jnp.float32)
        r = r + b_ref[...]
        if relu:
            r = jnp.maximum(r, 0.0)
        o_ref[...] = r

    return pl.pallas_call(
        body,
        grid=(N // bm,),
        in_specs=[
            pl.BlockSpec((bm, D), lambda i: (i, 0)),
            pl.BlockSpec((bm, D), lambda i: (i, 0)),
            pl.BlockSpec((D, D), lambda i: (0, 0)),
            pl.BlockSpec((1, D), lambda i: (0, 0)),
            pl.BlockSpec((D, D), lambda i: (0, 0)),
        ],
        out_specs=pl.BlockSpec((bm, D), lambda i: (i, 0)),
        out_shape=jax.ShapeDtypeStruct((N, D), jnp.float32),
    )(agg, xin, w_l, b_l.reshape(1, D), w_r)


def _slice_cols(m):
    return m.reshape(N, NW, CPW).transpose(1, 0, 2).reshape(NW, N * CPW)


def _unslice_cols(m):
    return m.reshape(NW, N, CPW).transpose(1, 0, 2).reshape(N, D)


def kernel(x, edge_index, W1_l, b1_l, W1_r, W2_l, b2_l, W2_r):
    src = edge_index[0].astype(jnp.int32)
    dst = edge_index[1].astype(jnp.int32)

    agg1 = _unslice_cols(_sc_segment_mean(_slice_cols(x), src, dst))
    h = _dense(agg1, x, W1_l, b1_l, W1_r, relu=True)

    agg2 = _unslice_cols(_sc_segment_mean(_slice_cols(h), src, dst))
    return _dense(agg2, h, W2_l, b2_l, W2_r, relu=False)


# speedup vs baseline: 7.4627x; 3.3226x over previous
import jax
import jax.numpy as jnp
from jax import lax
from jax.experimental import pallas as pl
from jax.experimental.pallas import tpu as pltpu
from jax.experimental.pallas import tpu_sc as plsc

N = 10000
E = 320000
D = 128
NC = 2
NS = 16
NW = NC * NS
CPW = D // NW
LANES = 16
CHUNK = 3200
NCHUNK = E // CHUNK
GROUPS = CHUNK // LANES


def _make_sc_body(with_deg):
    def body(table_hbm, edge_hbm, *refs):
        if with_deg:
            (out_hbm, deg_hbm, xt, acc, deg, eb_a, eb_b,
             sem_tab, sem_a, sem_b) = refs
        else:
            (out_hbm, xt, acc, eb_a, eb_b,
             sem_tab, sem_a, sem_b) = refs
            deg = None

        cid = lax.axis_index("c")
        sid = lax.axis_index("s")
        wid = sid * NC + cid

        base = wid * NCHUNK // NW

        def chunk_off(k):
            ck = base + k
            ck = jnp.where(ck >= NCHUNK, ck - NCHUNK, ck)
            return ck * CHUNK

        pltpu.make_async_copy(table_hbm.at[wid], xt, sem_tab).start()
        pltpu.make_async_copy(edge_hbm.at[pl.ds(chunk_off(0), CHUNK)], eb_a, sem_a).start()

        zero16 = jnp.zeros((LANES,), jnp.float32)
        ones16 = jnp.ones((LANES,), jnp.float32)

        def zero_acc(i, carry):
            acc[pl.ds(i * LANES, LANES)] = zero16
            return carry

        lax.fori_loop(0, (N * CPW) // LANES, zero_acc, None)

        if with_deg:
            def zero_deg(i, carry):
                deg[pl.ds(i * LANES, LANES)] = zero16
                return carry

            lax.fori_loop(0, N // LANES, zero_deg, None)

        pltpu.make_async_copy(table_hbm.at[wid], xt, sem_tab).wait()

        def process(eb):
            @plsc.parallel_loop(0, GROUPS, unroll=4)
            def grp(g):
                e16 = eb[pl.ds(g * LANES, LANES)]
                s16 = e16 & 0xFFFF
                d16 = lax.shift_right_logical(e16, 16)
                for c in range(CPW):
                    vals = plsc.load_gather(xt, [s16 + c * N])
                    plsc.addupdate_scatter(acc, [d16 + c * N], vals)
                if with_deg:
                    plsc.addupdate_scatter(deg, [d16], ones16)

        def chunk_pair(i, carry):
            pltpu.make_async_copy(edge_hbm.at[pl.ds(chunk_off(2 * i + 1), CHUNK)], eb_b, sem_b).start()
            pltpu.make_async_copy(edge_hbm.at[pl.ds(0, CHUNK)], eb_a, sem_a).wait()
            process(eb_a)
            pltpu.make_async_copy(edge_hbm.at[pl.ds(chunk_off(2 * i + 2), CHUNK)], eb_a, sem_a).start()
            pltpu.make_async_copy(edge_hbm.at[pl.ds(0, CHUNK)], eb_b, sem_b).wait()
            process(eb_b)
            return carry

        lax.fori_loop(0, NCHUNK // 2, chunk_pair, None)

        pltpu.make_async_copy(edge_hbm.at[pl.ds(0, CHUNK)], eb_a, sem_a).wait()

        pltpu.sync_copy(acc, out_hbm.at[wid])
        if with_deg:
            @pl.when(wid == 0)
            def _():
                pltpu.sync_copy(deg, deg_hbm)

    return body


def _sc_segment_sum(table_r, edges, with_deg):
    mesh = plsc.VectorSubcoreMesh(core_axis_name="c", subcore_axis_name="s")
    out_type = [jax.ShapeDtypeStruct((NW, N * CPW), jnp.float32)]
    scratch = [
        pltpu.VMEM((N * CPW,), jnp.float32),
        pltpu.VMEM((N * CPW,), jnp.float32),
        pltpu.VMEM((CHUNK,), jnp.int32),
        pltpu.VMEM((CHUNK,), jnp.int32),
        pltpu.SemaphoreType.DMA,
        pltpu.SemaphoreType.DMA,
        pltpu.SemaphoreType.DMA,
    ]
    if with_deg:
        out_type.append(jax.ShapeDtypeStruct((N,), jnp.float32))
        scratch.insert(2, pltpu.VMEM((N,), jnp.float32))
    f = pl.kernel(
        _make_sc_body(with_deg),
        out_type=tuple(out_type),
        mesh=mesh,
        compiler_params=pltpu.CompilerParams(needs_layout_passes=False),
        scratch_types=scratch,
    )
    return f(table_r, edges)


def _dense(agg_sum, deg, xin, w_l, b_l, w_r, relu):
    bm = 1000

    def body(a_ref, dg_ref, x_ref, wl_ref, b_ref, wr_ref, o_ref):
        r = jnp.dot(a_ref[...], wl_ref[...], preferred_element_type=jnp.float32)
        r = r / jnp.maximum(dg_ref[...], 1.0)
        r = r + jnp.dot(x_ref[...], wr_ref[...], preferred_element_type=jnp.float32)
        r = r + b_ref[...]
        if relu:
            r = jnp.maximum(r, 0.0)
        o_ref[...] = r

    return pl.pallas_call(
        body,
        grid=(N // bm,),
        in_specs=[
            pl.BlockSpec((bm, D), lambda i: (i, 0)),
            pl.BlockSpec((bm, 1), lambda i: (i, 0)),
            pl.BlockSpec((bm, D), lambda i: (i, 0)),
            pl.BlockSpec((D, D), lambda i: (0, 0)),
            pl.BlockSpec((1, D), lambda i: (0, 0)),
            pl.BlockSpec((D, D), lambda i: (0, 0)),
        ],
        out_specs=pl.BlockSpec((bm, D), lambda i: (i, 0)),
        out_shape=jax.ShapeDtypeStruct((N, D), jnp.float32),
    )(agg_sum, deg.reshape(N, 1), xin, w_l, b_l.reshape(1, D), w_r)


def _slice_cols(m):
    return m.reshape(N, NW, CPW).transpose(1, 2, 0).reshape(NW, CPW * N)


def _unslice_cols(m):
    return m.reshape(NW, CPW, N).transpose(2, 0, 1).reshape(N, D)


def kernel(x, edge_index, W1_l, b1_l, W1_r, W2_l, b2_l, W2_r):
    src = edge_index[0].astype(jnp.int32)
    dst = edge_index[1].astype(jnp.int32)
    edges = jnp.bitwise_or(jnp.left_shift(dst, 16), src)

    sum1, deg = _sc_segment_sum(_slice_cols(x), edges, with_deg=True)
    h = _dense(_unslice_cols(sum1), deg, x, W1_l, b1_l, W1_r, relu=True)

    (sum2,) = _sc_segment_sum(_slice_cols(h), edges, with_deg=False)
    return _dense(_unslice_cols(sum2), deg, h, W2_l, b2_l, W2_r, relu=False)


# speedup vs baseline: 7.4718x; 1.0012x over previous
import jax
import jax.numpy as jnp
from jax import lax
from jax.experimental import pallas as pl
from jax.experimental.pallas import tpu as pltpu
from jax.experimental.pallas import tpu_sc as plsc

N = 10000
E = 320000
D = 128
NC = 2
NS = 16
NW = NC * NS
CPW = D // NW
LANES = 16
CHUNK = 3200
NCHUNK = E // CHUNK
GROUPS = CHUNK // LANES


def _make_sc_body(with_deg):
    def body(table_hbm, edge_hbm, *refs):
        if with_deg:
            (out_hbm, deg_hbm, xt, acc, deg, eb_a, eb_b,
             sem_tab, sem_a, sem_b) = refs
        else:
            (out_hbm, xt, acc, eb_a, eb_b,
             sem_tab, sem_a, sem_b) = refs
            deg = None

        cid = lax.axis_index("c")
        sid = lax.axis_index("s")
        wid = sid * NC + cid

        base = wid * NCHUNK // NW

        def chunk_off(k):
            ck = base + k
            ck = jnp.where(ck >= NCHUNK, ck - NCHUNK, ck)
            return ck * CHUNK

        pltpu.make_async_copy(table_hbm.at[wid], xt, sem_tab).start()
        pltpu.make_async_copy(edge_hbm.at[pl.ds(chunk_off(0), CHUNK)], eb_a, sem_a).start()

        zero16 = jnp.zeros((LANES,), jnp.float32)
        ones16 = jnp.ones((LANES,), jnp.float32)

        def zero_acc(i, carry):
            acc[pl.ds(i * LANES, LANES)] = zero16
            return carry

        lax.fori_loop(0, (N * CPW) // LANES, zero_acc, None)

        if with_deg:
            def zero_deg(i, carry):
                deg[pl.ds(i * LANES, LANES)] = zero16
                return carry

            lax.fori_loop(0, N // LANES, zero_deg, None)

        pltpu.make_async_copy(table_hbm.at[wid], xt, sem_tab).wait()

        def process(eb):
            @plsc.parallel_loop(0, GROUPS, unroll=4)
            def grp(g):
                e16 = eb[pl.ds(g * LANES, LANES)]
                s16 = e16 & 0xFFFF
                d16 = lax.shift_right_logical(e16, 16)
                for c in range(CPW):
                    vals = plsc.load_gather(xt, [s16 + c * N])
                    plsc.addupdate_scatter(acc, [d16 + c * N], vals)
                if with_deg:
                    plsc.addupdate_scatter(deg, [d16], ones16)

        def chunk_pair(i, carry):
            pltpu.make_async_copy(edge_hbm.at[pl.ds(chunk_off(2 * i + 1), CHUNK)], eb_b, sem_b).start()
            pltpu.make_async_copy(edge_hbm.at[pl.ds(0, CHUNK)], eb_a, sem_a).wait()
            process(eb_a)
            pltpu.make_async_copy(edge_hbm.at[pl.ds(chunk_off(2 * i + 2), CHUNK)], eb_a, sem_a).start()
            pltpu.make_async_copy(edge_hbm.at[pl.ds(0, CHUNK)], eb_b, sem_b).wait()
            process(eb_b)
            return carry

        lax.fori_loop(0, NCHUNK // 2, chunk_pair, None)

        pltpu.make_async_copy(edge_hbm.at[pl.ds(0, CHUNK)], eb_a, sem_a).wait()

        pltpu.sync_copy(acc, out_hbm.at[wid])
        if with_deg:
            @pl.when(wid == 0)
            def _():
                pltpu.sync_copy(deg, deg_hbm)

    return body


def _sc_segment_sum(table_r, edges, with_deg):
    mesh = plsc.VectorSubcoreMesh(core_axis_name="c", subcore_axis_name="s")
    out_type = [jax.ShapeDtypeStruct((NW, N * CPW), jnp.float32)]
    scratch = [
        pltpu.VMEM((N * CPW,), jnp.float32),
        pltpu.VMEM((N * CPW,), jnp.float32),
        pltpu.VMEM((CHUNK,), jnp.int32),
        pltpu.VMEM((CHUNK,), jnp.int32),
        pltpu.SemaphoreType.DMA,
        pltpu.SemaphoreType.DMA,
        pltpu.SemaphoreType.DMA,
    ]
    if with_deg:
        out_type.append(jax.ShapeDtypeStruct((N,), jnp.float32))
        scratch.insert(2, pltpu.VMEM((N,), jnp.float32))
    f = pl.kernel(
        _make_sc_body(with_deg),
        out_type=tuple(out_type),
        mesh=mesh,
        compiler_params=pltpu.CompilerParams(needs_layout_passes=False),
        scratch_types=scratch,
    )
    return f(table_r, edges)


def _matmul_bias(xin, w, b):
    bm = 1000

    def body(x_ref, w_ref, b_ref, o_ref):
        o_ref[...] = (
            jnp.dot(x_ref[...], w_ref[...], preferred_element_type=jnp.float32)
            + b_ref[...])

    return pl.pallas_call(
        body,
        grid=(N // bm,),
        in_specs=[
            pl.BlockSpec((bm, D), lambda i: (i, 0)),
            pl.BlockSpec((D, D), lambda i: (0, 0)),
            pl.BlockSpec((1, D), lambda i: (0, 0)),
        ],
        out_specs=pl.BlockSpec((bm, D), lambda i: (i, 0)),
        out_shape=jax.ShapeDtypeStruct((N, D), jnp.float32),
    )(xin, w, b.reshape(1, D))


def _combine(agg_sum, deg, part, w_l, relu):
    bm = 1000

    def body(a_ref, dg_ref, p_ref, wl_ref, o_ref):
        r = jnp.dot(a_ref[...], wl_ref[...], preferred_element_type=jnp.float32)
        r = r / jnp.maximum(dg_ref[...], 1.0)
        r = r + p_ref[...]
        if relu:
            r = jnp.maximum(r, 0.0)
        o_ref[...] = r

    return pl.pallas_call(
        body,
        grid=(N // bm,),
        in_specs=[
            pl.BlockSpec((bm, D), lambda i: (i, 0)),
            pl.BlockSpec((bm, 1), lambda i: (i, 0)),
            pl.BlockSpec((bm, D), lambda i: (i, 0)),
            pl.BlockSpec((D, D), lambda i: (0, 0)),
        ],
        out_specs=pl.BlockSpec((bm, D), lambda i: (i, 0)),
        out_shape=jax.ShapeDtypeStruct((N, D), jnp.float32),
    )(agg_sum, deg.reshape(N, 1), part, w_l)


def _slice_cols(m):
    return m.reshape(N, NW, CPW).transpose(1, 2, 0).reshape(NW, CPW * N)


def _unslice_cols(m):
    return m.reshape(NW, CPW, N).transpose(2, 0, 1).reshape(N, D)


def kernel(x, edge_index, W1_l, b1_l, W1_r, W2_l, b2_l, W2_r):
    src = edge_index[0].astype(jnp.int32)
    dst = edge_index[1].astype(jnp.int32)
    edges = jnp.bitwise_or(jnp.left_shift(dst, 16), src)

    sum1, deg = _sc_segment_sum(_slice_cols(x), edges, with_deg=True)
    p1 = _matmul_bias(x, W1_r, b1_l)
    h = _combine(_unslice_cols(sum1), deg, p1, W1_l, relu=True)

    (sum2,) = _sc_segment_sum(_slice_cols(h), edges, with_deg=False)
    p2 = _matmul_bias(h, W2_r, b2_l)
    return _combine(_unslice_cols(sum2), deg, p2, W2_l, relu=False)


# speedup vs baseline: 7.7502x; 1.0373x over previous
import jax
import jax.numpy as jnp
from jax import lax
from jax.experimental import pallas as pl
from jax.experimental.pallas import tpu as pltpu
from jax.experimental.pallas import tpu_sc as plsc

N = 10000
E = 320000
D = 128
NC = 2
NS = 16
NW = NC * NS
CPW = D // NW
LANES = 16
CHUNK = 6400
NCHUNK = E // CHUNK
GROUPS = CHUNK // LANES


def _make_sc_body(with_deg):
    def body(table_hbm, edge_hbm, *refs):
        if with_deg:
            (out_hbm, deg_hbm, xt, acc, deg, eb_a, eb_b,
             sem_tab, sem_a, sem_b) = refs
        else:
            (out_hbm, xt, acc, eb_a, eb_b,
             sem_tab, sem_a, sem_b) = refs
            deg = None

        cid = lax.axis_index("c")
        sid = lax.axis_index("s")
        wid = sid * NC + cid

        base = wid * NCHUNK // NW

        def chunk_off(k):
            ck = base + k
            ck = jnp.where(ck >= NCHUNK, ck - NCHUNK, ck)
            return ck * CHUNK

        pltpu.make_async_copy(table_hbm.at[wid], xt, sem_tab).start()
        pltpu.make_async_copy(edge_hbm.at[pl.ds(chunk_off(0), CHUNK)], eb_a, sem_a).start()

        zero16 = jnp.zeros((LANES,), jnp.float32)
        ones16 = jnp.ones((LANES,), jnp.float32)

        @plsc.parallel_loop(0, (N * CPW) // LANES, unroll=4)
        def zero_acc(i):
            acc[pl.ds(i * LANES, LANES)] = zero16

        if with_deg:
            @plsc.parallel_loop(0, N // LANES, unroll=4)
            def zero_deg(i):
                deg[pl.ds(i * LANES, LANES)] = zero16

        pltpu.make_async_copy(table_hbm.at[wid], xt, sem_tab).wait()

        def process(eb):
            @plsc.parallel_loop(0, GROUPS, unroll=4)
            def grp(g):
                e16 = eb[pl.ds(g * LANES, LANES)]
                s16 = e16 & 0xFFFF
                d16 = lax.shift_right_logical(e16, 16)
                for c in range(CPW):
                    vals = plsc.load_gather(xt, [s16 + c * N])
                    plsc.addupdate_scatter(acc, [d16 + c * N], vals)
                if with_deg:
                    plsc.addupdate_scatter(deg, [d16], ones16)

        def chunk_pair(i, carry):
            pltpu.make_async_copy(edge_hbm.at[pl.ds(chunk_off(2 * i + 1), CHUNK)], eb_b, sem_b).start()
            pltpu.make_async_copy(edge_hbm.at[pl.ds(0, CHUNK)], eb_a, sem_a).wait()
            process(eb_a)
            pltpu.make_async_copy(edge_hbm.at[pl.ds(chunk_off(2 * i + 2), CHUNK)], eb_a, sem_a).start()
            pltpu.make_async_copy(edge_hbm.at[pl.ds(0, CHUNK)], eb_b, sem_b).wait()
            process(eb_b)
            return carry

        lax.fori_loop(0, NCHUNK // 2, chunk_pair, None)

        pltpu.make_async_copy(edge_hbm.at[pl.ds(0, CHUNK)], eb_a, sem_a).wait()

        pltpu.sync_copy(acc, out_hbm.at[wid])
        if with_deg:
            @pl.when(wid == 0)
            def _():
                pltpu.sync_copy(deg, deg_hbm)

    return body


def _sc_segment_sum(table_r, edges, with_deg):
    mesh = plsc.VectorSubcoreMesh(core_axis_name="c", subcore_axis_name="s")
    out_type = [jax.ShapeDtypeStruct((NW, N * CPW), jnp.float32)]
    scratch = [
        pltpu.VMEM((N * CPW,), jnp.float32),
        pltpu.VMEM((N * CPW,), jnp.float32),
        pltpu.VMEM((CHUNK,), jnp.int32),
        pltpu.VMEM((CHUNK,), jnp.int32),
        pltpu.SemaphoreType.DMA,
        pltpu.SemaphoreType.DMA,
        pltpu.SemaphoreType.DMA,
    ]
    if with_deg:
        out_type.append(jax.ShapeDtypeStruct((N,), jnp.float32))
        scratch.insert(2, pltpu.VMEM((N,), jnp.float32))
    f = pl.kernel(
        _make_sc_body(with_deg),
        out_type=tuple(out_type),
        mesh=mesh,
        compiler_params=pltpu.CompilerParams(needs_layout_passes=False),
        scratch_types=scratch,
    )
    return f(table_r, edges)


def _matmul_bias(xin, w, b):
    bm = 1000

    def body(x_ref, w_ref, b_ref, o_ref):
        o_ref[...] = (
            jnp.dot(x_ref[...], w_ref[...], preferred_element_type=jnp.float32)
            + b_ref[...])

    return pl.pallas_call(
        body,
        grid=(N // bm,),
        in_specs=[
            pl.BlockSpec((bm, D), lambda i: (i, 0)),
            pl.BlockSpec((D, D), lambda i: (0, 0)),
            pl.BlockSpec((1, D), lambda i: (0, 0)),
        ],
        out_specs=pl.BlockSpec((bm, D), lambda i: (i, 0)),
        out_shape=jax.ShapeDtypeStruct((N, D), jnp.float32),
    )(xin, w, b.reshape(1, D))


def _combine(agg_sum, deg, part, w_l, relu):
    bm = 1000

    def body(a_ref, dg_ref, p_ref, wl_ref, o_ref):
        r = jnp.dot(a_ref[...], wl_ref[...], preferred_element_type=jnp.float32)
        r = r / jnp.maximum(dg_ref[...], 1.0)
        r = r + p_ref[...]
        if relu:
            r = jnp.maximum(r, 0.0)
        o_ref[...] = r

    return pl.pallas_call(
        body,
        grid=(N // bm,),
        in_specs=[
            pl.BlockSpec((bm, D), lambda i: (i, 0)),
            pl.BlockSpec((bm, 1), lambda i: (i, 0)),
            pl.BlockSpec((bm, D), lambda i: (i, 0)),
            pl.BlockSpec((D, D), lambda i: (0, 0)),
        ],
        out_specs=pl.BlockSpec((bm, D), lambda i: (i, 0)),
        out_shape=jax.ShapeDtypeStruct((N, D), jnp.float32),
    )(agg_sum, deg.reshape(N, 1), part, w_l)


def _slice_cols(m):
    return m.reshape(N, NW, CPW).transpose(1, 2, 0).reshape(NW, CPW * N)


def _unslice_cols(m):
    return m.reshape(NW, CPW, N).transpose(2, 0, 1).reshape(N, D)


def kernel(x, edge_index, W1_l, b1_l, W1_r, W2_l, b2_l, W2_r):
    src = edge_index[0].astype(jnp.int32)
    dst = edge_index[1].astype(jnp.int32)
    edges = jnp.bitwise_or(jnp.left_shift(dst, 16), src)

    sum1, deg = _sc_segment_sum(_slice_cols(x), edges, with_deg=True)
    p1 = _matmul_bias(x, W1_r, b1_l)
    h = _combine(_unslice_cols(sum1), deg, p1, W1_l, relu=True)

    (sum2,) = _sc_segment_sum(_slice_cols(h), edges, with_deg=False)
    p2 = _matmul_bias(h, W2_r, b2_l)
    return _combine(_unslice_cols(sum2), deg, p2, W2_l, relu=False)


# speedup vs baseline: 7.7972x; 1.0061x over previous
import jax
import jax.numpy as jnp
from jax import lax
from jax.experimental import pallas as pl
from jax.experimental.pallas import tpu as pltpu
from jax.experimental.pallas import tpu_sc as plsc

N = 10000
E = 320000
D = 128
NC = 2
NS = 16
NW = NC * NS
CPW = D // NW
LANES = 16
CHUNK = 8000
NCHUNK = E // CHUNK
GROUPS = CHUNK // LANES


def _make_sc_body(with_deg):
    def body(table_hbm, edge_hbm, *refs):
        if with_deg:
            (out_hbm, deg_hbm, xt, acc, deg, eb_a, eb_b,
             sem_tab, sem_a, sem_b) = refs
        else:
            (out_hbm, xt, acc, eb_a, eb_b,
             sem_tab, sem_a, sem_b) = refs
            deg = None

        cid = lax.axis_index("c")
        sid = lax.axis_index("s")
        wid = sid * NC + cid

        base = wid * NCHUNK // NW

        def chunk_off(k):
            ck = base + k
            ck = jnp.where(ck >= NCHUNK, ck - NCHUNK, ck)
            return ck * CHUNK

        pltpu.make_async_copy(table_hbm.at[wid], xt, sem_tab).start()
        pltpu.make_async_copy(edge_hbm.at[pl.ds(chunk_off(0), CHUNK)], eb_a, sem_a).start()

        zero16 = jnp.zeros((LANES,), jnp.float32)
        ones16 = jnp.ones((LANES,), jnp.float32)

        @plsc.parallel_loop(0, (N * CPW) // LANES, unroll=4)
        def zero_acc(i):
            acc[pl.ds(i * LANES, LANES)] = zero16

        if with_deg:
            @plsc.parallel_loop(0, N // LANES, unroll=4)
            def zero_deg(i):
                deg[pl.ds(i * LANES, LANES)] = zero16

        pltpu.make_async_copy(table_hbm.at[wid], xt, sem_tab).wait()

        def process(eb):
            @plsc.parallel_loop(0, GROUPS, unroll=4)
            def grp(g):
                e16 = eb[pl.ds(g * LANES, LANES)]
                s16 = e16 & 0xFFFF
                d16 = lax.shift_right_logical(e16, 16)
                for c in range(CPW):
                    vals = plsc.load_gather(xt, [s16 + c * N])
                    plsc.addupdate_scatter(acc, [d16 + c * N], vals)
                if with_deg:
                    plsc.addupdate_scatter(deg, [d16], ones16)

        def chunk_pair(i, carry):
            pltpu.make_async_copy(edge_hbm.at[pl.ds(chunk_off(2 * i + 1), CHUNK)], eb_b, sem_b).start()
            pltpu.make_async_copy(edge_hbm.at[pl.ds(0, CHUNK)], eb_a, sem_a).wait()
            process(eb_a)
            pltpu.make_async_copy(edge_hbm.at[pl.ds(chunk_off(2 * i + 2), CHUNK)], eb_a, sem_a).start()
            pltpu.make_async_copy(edge_hbm.at[pl.ds(0, CHUNK)], eb_b, sem_b).wait()
            process(eb_b)
            return carry

        lax.fori_loop(0, NCHUNK // 2, chunk_pair, None)

        pltpu.make_async_copy(edge_hbm.at[pl.ds(0, CHUNK)], eb_a, sem_a).wait()

        pltpu.sync_copy(acc, out_hbm.at[wid])
        if with_deg:
            @pl.when(wid == 0)
            def _():
                pltpu.sync_copy(deg, deg_hbm)

    return body


def _sc_segment_sum(table_r, edges, with_deg):
    mesh = plsc.VectorSubcoreMesh(core_axis_name="c", subcore_axis_name="s")
    out_type = [jax.ShapeDtypeStruct((NW, N * CPW), jnp.float32)]
    scratch = [
        pltpu.VMEM((N * CPW,), jnp.float32),
        pltpu.VMEM((N * CPW,), jnp.float32),
        pltpu.VMEM((CHUNK,), jnp.int32),
        pltpu.VMEM((CHUNK,), jnp.int32),
        pltpu.SemaphoreType.DMA,
        pltpu.SemaphoreType.DMA,
        pltpu.SemaphoreType.DMA,
    ]
    if with_deg:
        out_type.append(jax.ShapeDtypeStruct((N,), jnp.float32))
        scratch.insert(2, pltpu.VMEM((N,), jnp.float32))
    f = pl.kernel(
        _make_sc_body(with_deg),
        out_type=tuple(out_type),
        mesh=mesh,
        compiler_params=pltpu.CompilerParams(needs_layout_passes=False),
        scratch_types=scratch,
    )
    return f(table_r, edges)


def _matmul_bias(xin, w, b):
    bm = 2000

    def body(x_ref, w_ref, b_ref, o_ref):
        o_ref[...] = (
            jnp.dot(x_ref[...], w_ref[...], preferred_element_type=jnp.float32)
            + b_ref[...])

    return pl.pallas_call(
        body,
        grid=(N // bm,),
        in_specs=[
            pl.BlockSpec((bm, D), lambda i: (i, 0)),
            pl.BlockSpec((D, D), lambda i: (0, 0)),
            pl.BlockSpec((1, D), lambda i: (0, 0)),
        ],
        out_specs=pl.BlockSpec((bm, D), lambda i: (i, 0)),
        out_shape=jax.ShapeDtypeStruct((N, D), jnp.float32),
    )(xin, w, b.reshape(1, D))


def _combine(agg_sum, deg, part, w_l, relu):
    bm = 2000

    def body(a_ref, dg_ref, p_ref, wl_ref, o_ref):
        r = jnp.dot(a_ref[...], wl_ref[...], preferred_element_type=jnp.float32)
        r = r / jnp.maximum(dg_ref[...], 1.0)
        r = r + p_ref[...]
        if relu:
            r = jnp.maximum(r, 0.0)
        o_ref[...] = r

    return pl.pallas_call(
        body,
        grid=(N // bm,),
        in_specs=[
            pl.BlockSpec((bm, D), lambda i: (i, 0)),
            pl.BlockSpec((bm, 1), lambda i: (i, 0)),
            pl.BlockSpec((bm, D), lambda i: (i, 0)),
            pl.BlockSpec((D, D), lambda i: (0, 0)),
        ],
        out_specs=pl.BlockSpec((bm, D), lambda i: (i, 0)),
        out_shape=jax.ShapeDtypeStruct((N, D), jnp.float32),
    )(agg_sum, deg.reshape(N, 1), part, w_l)


def _slice_cols(m):
    return m.reshape(N, NW, CPW).transpose(1, 2, 0).reshape(NW, CPW * N)


def _unslice_cols(m):
    return m.reshape(NW, CPW, N).transpose(2, 0, 1).reshape(N, D)


def kernel(x, edge_index, W1_l, b1_l, W1_r, W2_l, b2_l, W2_r):
    src = edge_index[0].astype(jnp.int32)
    dst = edge_index[1].astype(jnp.int32)
    edges = jnp.bitwise_or(jnp.left_shift(dst, 16), src)

    sum1, deg = _sc_segment_sum(_slice_cols(x), edges, with_deg=True)
    p1 = _matmul_bias(x, W1_r, b1_l)
    h = _combine(_unslice_cols(sum1), deg, p1, W1_l, relu=True)

    (sum2,) = _sc_segment_sum(_slice_cols(h), edges, with_deg=False)
    p2 = _matmul_bias(h, W2_r, b2_l)
    return _combine(_unslice_cols(sum2), deg, p2, W2_l, relu=False)
